# Initial kernel scaffold; baseline (speedup 1.0000x reference)
import functools

import jax
import jax.numpy as jnp
from jax import lax
from jax.experimental import pallas as pl
from jax.experimental.pallas import tpu as pltpu
from jax.experimental.pallas import tpu_sc as plsc

N = 100000
E = 3200000
D_EDGE = 16

NC = 2
NS = 16
NW = NC * NS

CHUNK = 1024
SUB = 128
KSUB = CHUNK // SUB
NCHUNKS = E // CHUNK
ROWS_PER_TILE = N // NS

_BASE_CHUNKS = NCHUNKS // NW
_EXTRA_WORKERS = NCHUNKS % NW


def _seg_sum_body(zeros_hbm, idx_hbm, attr_hbm, out_hbm, acc, idxbuf, databuf,
                  sem_i, sem_a, sem_s):
    c = lax.axis_index("c")
    s = lax.axis_index("s")
    wid = s * NC + c

    rbase = s * ROWS_PER_TILE
    pltpu.sync_copy(zeros_hbm.at[pl.ds(rbase, ROWS_PER_TILE)],
                    acc.at[pl.ds(rbase, ROWS_PER_TILE)])
    plsc.subcore_barrier()

    nchunks_w = _BASE_CHUNKS + jnp.where(wid < _EXTRA_WORKERS, 1, 0)

    def body(j, carry):
        chunk = wid + NW * j
        cp_i = pltpu.async_copy(idx_hbm.at[chunk], idxbuf, sem_i)
        cp_a = pltpu.async_copy(attr_hbm.at[pl.ds(chunk * CHUNK, CHUNK)],
                                databuf, sem_a)
        cp_i.wait()
        cp_a.wait()
        cps = []
        for k in range(KSUB):
            cps.append(pltpu.async_copy(
                databuf.at[pl.ds(k * SUB, SUB)],
                acc.at[idxbuf.at[k]],
                sem_s, add=True))
        for cp in cps:
            cp.wait()
        return carry

    lax.fori_loop(0, nchunks_w, body, 0, unroll=False)

    plsc.subcore_barrier()
    pltpu.sync_copy(acc.at[pl.ds(rbase, ROWS_PER_TILE)],
                    out_hbm.at[c, pl.ds(rbase, ROWS_PER_TILE)])


def _sc_segment_sum(row3d, edge_attr):
    zeros = jnp.zeros((N, D_EDGE), dtype=jnp.float32)
    mesh = plsc.VectorSubcoreMesh(core_axis_name="c", subcore_axis_name="s")
    k = pl.kernel(
        _seg_sum_body,
        out_type=jax.ShapeDtypeStruct((NC, N, D_EDGE), jnp.float32),
        mesh=mesh,
        scratch_types=[
            pltpu.VMEM_SHARED((N, D_EDGE), jnp.float32),
            pltpu.VMEM((KSUB, SUB), jnp.int32),
            pltpu.VMEM((CHUNK, D_EDGE), jnp.float32),
            pltpu.SemaphoreType.DMA,
            pltpu.SemaphoreType.DMA,
            pltpu.SemaphoreType.DMA,
        ],
    )
    return k(zeros, row3d, edge_attr)


BN = 2000


def _mlp_body(x_ref, p_ref, w1a_ref, w1b_ref, b1_ref, w2_ref, b2_ref, o_ref):
    agg = p_ref[0] + p_ref[1]
    h = jnp.dot(x_ref[...], w1a_ref[...], preferred_element_type=jnp.float32)
    h += jnp.dot(agg, w1b_ref[...], preferred_element_type=jnp.float32)
    h = jnp.maximum(h + b1_ref[...], 0.0)
    o = jnp.dot(h, w2_ref[...], preferred_element_type=jnp.float32)
    o_ref[...] = o + b2_ref[...]


def _tc_mlp(node_feats, partials, W1, b1, W2, b2):
    d_feat = node_feats.shape[1]
    w1a = W1[:, :d_feat].T
    w1b = W1[:, d_feat:].T
    w2 = W2.T
    b1r = b1.reshape(1, -1)
    b2r = b2.reshape(1, -1)
    grid = (N // BN,)
    return pl.pallas_call(
        _mlp_body,
        grid=grid,
        in_specs=[
            pl.BlockSpec((BN, d_feat), lambda i: (i, 0)),
            pl.BlockSpec((NC, BN, D_EDGE), lambda i: (0, i, 0)),
            pl.BlockSpec(w1a.shape, lambda i: (0, 0)),
            pl.BlockSpec(w1b.shape, lambda i: (0, 0)),
            pl.BlockSpec((1, b1r.shape[1]), lambda i: (0, 0)),
            pl.BlockSpec(w2.shape, lambda i: (0, 0)),
            pl.BlockSpec((1, b2r.shape[1]), lambda i: (0, 0)),
        ],
        out_specs=pl.BlockSpec((BN, W2.shape[0]), lambda i: (i, 0)),
        out_shape=jax.ShapeDtypeStruct((N, W2.shape[0]), jnp.float32),
    )(node_feats, partials, w1a, w1b, b1r, w2, b2r)


def kernel(node_feats, edge_index, edge_attr, W1, b1, W2, b2):
    row3d = edge_index[0].reshape(NCHUNKS, KSUB, SUB)
    partials = _sc_segment_sum(row3d, edge_attr)
    return _tc_mlp(node_feats, partials, W1, b1, W2, b2)


# speedup vs baseline: 6.1793x; 6.1793x over previous
import functools

import jax
import jax.numpy as jnp
from jax import lax
from jax.experimental import pallas as pl
from jax.experimental.pallas import tpu as pltpu
from jax.experimental.pallas import tpu_sc as plsc

N = 100000
E = 3200000
D_EDGE = 16

NC = 2
NS = 16
NW = NC * NS

CHUNK = 1024
SUB = 128
KSUB = CHUNK // SUB
NCHUNKS = E // CHUNK
ROWS_A = 6256
ROWS_LAST = N - (NS - 1) * ROWS_A

_BASE_CHUNKS = NCHUNKS // NW
_EXTRA_WORKERS = NCHUNKS % NW


def _seg_sum_body(zeros_hbm, idx_hbm, attr_hbm, out_hbm, acc, idxbuf, databuf,
                  sem_i, sem_a, sem_s):
    c = lax.axis_index("c")
    s = lax.axis_index("s")
    wid = s * NC + c

    rbase = s * ROWS_A

    @pl.when(s < NS - 1)
    def _():
        pltpu.sync_copy(zeros_hbm.at[pl.ds(rbase, ROWS_A)],
                        acc.at[pl.ds(rbase, ROWS_A)])

    @pl.when(s == NS - 1)
    def _():
        pltpu.sync_copy(zeros_hbm.at[pl.ds((NS - 1) * ROWS_A, ROWS_LAST)],
                        acc.at[pl.ds((NS - 1) * ROWS_A, ROWS_LAST)])

    plsc.subcore_barrier()

    nchunks_w = _BASE_CHUNKS + jnp.where(wid < _EXTRA_WORKERS, 1, 0)

    def body(j, carry):
        chunk = wid + NW * j
        cp_i = pltpu.async_copy(idx_hbm.at[chunk], idxbuf, sem_i)
        cp_a = pltpu.async_copy(attr_hbm.at[pl.ds(chunk * CHUNK, CHUNK)],
                                databuf, sem_a)
        cp_i.wait()
        cp_a.wait()
        cps = []
        for k in range(KSUB):
            cps.append(pltpu.async_copy(
                databuf.at[pl.ds(k * SUB, SUB)],
                acc.at[idxbuf.at[k]],
                sem_s, add=True))
        for cp in cps:
            cp.wait()
        return carry

    lax.fori_loop(0, nchunks_w, body, 0, unroll=False)

    plsc.subcore_barrier()

    @pl.when(s < NS - 1)
    def _():
        pltpu.sync_copy(acc.at[pl.ds(rbase, ROWS_A)],
                        out_hbm.at[c, pl.ds(rbase, ROWS_A)])

    @pl.when(s == NS - 1)
    def _():
        pltpu.sync_copy(acc.at[pl.ds((NS - 1) * ROWS_A, ROWS_LAST)],
                        out_hbm.at[c, pl.ds((NS - 1) * ROWS_A, ROWS_LAST)])


def _sc_segment_sum(row3d, edge_attr):
    zeros = jnp.zeros((N, D_EDGE), dtype=jnp.float32)
    mesh = plsc.VectorSubcoreMesh(core_axis_name="c", subcore_axis_name="s")
    k = pl.kernel(
        _seg_sum_body,
        out_type=jax.ShapeDtypeStruct((NC, N, D_EDGE), jnp.float32),
        mesh=mesh,
        scratch_types=[
            pltpu.VMEM_SHARED((N, D_EDGE), jnp.float32),
            pltpu.VMEM((KSUB, SUB), jnp.int32),
            pltpu.VMEM((CHUNK, D_EDGE), jnp.float32),
            pltpu.SemaphoreType.DMA,
            pltpu.SemaphoreType.DMA,
            pltpu.SemaphoreType.DMA,
        ],
        compiler_params=pltpu.CompilerParams(use_tc_tiling_on_sc=False),
    )
    return k(zeros, row3d, edge_attr)


BN = 2000


def _mlp_body(x_ref, p_ref, w1a_ref, w1b_ref, b1_ref, w2_ref, b2_ref, o_ref):
    agg = p_ref[0] + p_ref[1]
    h = jnp.dot(x_ref[...], w1a_ref[...], preferred_element_type=jnp.float32)
    h += jnp.dot(agg, w1b_ref[...], preferred_element_type=jnp.float32)
    h = jnp.maximum(h + b1_ref[...], 0.0)
    o = jnp.dot(h, w2_ref[...], preferred_element_type=jnp.float32)
    o_ref[...] = o + b2_ref[...]


def _tc_mlp(node_feats, partials, W1, b1, W2, b2):
    d_feat = node_feats.shape[1]
    w1a = W1[:, :d_feat].T
    w1b = W1[:, d_feat:].T
    w2 = W2.T
    b1r = b1.reshape(1, -1)
    b2r = b2.reshape(1, -1)
    grid = (N // BN,)
    return pl.pallas_call(
        _mlp_body,
        grid=grid,
        in_specs=[
            pl.BlockSpec((BN, d_feat), lambda i: (i, 0)),
            pl.BlockSpec((NC, BN, D_EDGE), lambda i: (0, i, 0)),
            pl.BlockSpec(w1a.shape, lambda i: (0, 0)),
            pl.BlockSpec(w1b.shape, lambda i: (0, 0)),
            pl.BlockSpec((1, b1r.shape[1]), lambda i: (0, 0)),
            pl.BlockSpec(w2.shape, lambda i: (0, 0)),
            pl.BlockSpec((1, b2r.shape[1]), lambda i: (0, 0)),
        ],
        out_specs=pl.BlockSpec((BN, W2.shape[0]), lambda i: (i, 0)),
        out_shape=jax.ShapeDtypeStruct((N, W2.shape[0]), jnp.float32),
    )(node_feats, partials, w1a, w1b, b1r, w2, b2r)


def kernel(node_feats, edge_index, edge_attr, W1, b1, W2, b2):
    row3d = edge_index[0].reshape(NCHUNKS, KSUB, SUB)
    partials = _sc_segment_sum(row3d, edge_attr)
    return _tc_mlp(node_feats, partials, W1, b1, W2, b2)


# speedup vs baseline: 15.7019x; 2.5410x over previous
import functools

import jax
import jax.numpy as jnp
from jax import lax
from jax.experimental import pallas as pl
from jax.experimental.pallas import tpu as pltpu
from jax.experimental.pallas import tpu_sc as plsc

N = 100000
E = 3200000
D_EDGE = 16

NC = 2
NS = 16
NW = NC * NS

EB = 128
NEB = E // EB
SUB = 128
KSUB = 4
CHUNK = KSUB * EB
NCHUNKS = E // CHUNK
L = 16

ROWS_A = 6256
ROWS_LAST = N - (NS - 1) * ROWS_A

_BASE_CHUNKS = NCHUNKS // NW
_EXTRA_WORKERS = NCHUNKS % NW


def _seg_sum_body(zeros_hbm, idx_hbm, attr_hbm, out_hbm, acc, idxbuf, vlo, vhi,
                  rowbuf, sem_i, sem_lo, sem_hi, sem_s):
    c = lax.axis_index("c")
    s = lax.axis_index("s")
    wid = s * NC + c

    rbase = s * ROWS_A

    @pl.when(s < NS - 1)
    def _():
        pltpu.sync_copy(zeros_hbm.at[pl.ds(rbase, ROWS_A)],
                        acc.at[pl.ds(rbase, ROWS_A)])

    @pl.when(s == NS - 1)
    def _():
        pltpu.sync_copy(zeros_hbm.at[pl.ds((NS - 1) * ROWS_A, ROWS_LAST)],
                        acc.at[pl.ds((NS - 1) * ROWS_A, ROWS_LAST)])

    plsc.subcore_barrier()

    nchunks_w = _BASE_CHUNKS + jnp.where(wid < _EXTRA_WORKERS, 1, 0)
    iota16 = lax.iota(jnp.int32, L)

    def start_loads(j, b):
        chunk = wid + NW * j
        r0 = chunk * KSUB * 8
        pltpu.async_copy(idx_hbm.at[chunk], idxbuf.at[b], sem_i)
        pltpu.async_copy(attr_hbm.at[0, pl.ds(r0, KSUB * 8)], vlo.at[b], sem_lo)
        pltpu.async_copy(attr_hbm.at[1, pl.ds(r0, KSUB * 8)], vhi.at[b], sem_hi)

    def wait_loads(j, b):
        chunk = wid + NW * j
        r0 = chunk * KSUB * 8
        pltpu.make_async_copy(idx_hbm.at[chunk], idxbuf.at[b], sem_i).wait()
        pltpu.make_async_copy(attr_hbm.at[0, pl.ds(r0, KSUB * 8)], vlo.at[b],
                              sem_lo).wait()
        pltpu.make_async_copy(attr_hbm.at[1, pl.ds(r0, KSUB * 8)], vhi.at[b],
                              sem_hi).wait()

    start_loads(0, 0)

    def body(j, carry):
        b = lax.rem(j, 2)
        wait_loads(j, b)

        @pl.when(j + 1 < nchunks_w)
        def _():
            start_loads(j + 1, 1 - b)

        def trans_sb(sb, carry2):
            ebase = sb * EB
            for fi in range(8):
                fvec_lo = jnp.full((L,), fi, dtype=jnp.int32)
                fvec_hi = jnp.full((L,), fi + 8, dtype=jnp.int32)
                for g in range(EB // L):
                    evec = iota16 + (ebase + g * L)
                    v = vlo[b, sb * 8 + fi, pl.ds(g * L, L)]
                    plsc.store_scatter(rowbuf, [evec, fvec_lo], v)
                    v = vhi[b, sb * 8 + fi, pl.ds(g * L, L)]
                    plsc.store_scatter(rowbuf, [evec, fvec_hi], v)
            return carry2

        lax.fori_loop(0, KSUB, trans_sb, 0, unroll=False)

        cps = []
        for k in range(KSUB):
            cps.append(pltpu.async_copy(
                rowbuf.at[pl.ds(k * SUB, SUB)],
                acc.at[idxbuf.at[b, k]],
                sem_s, add=True))
        for cp in cps:
            cp.wait()
        return carry

    lax.fori_loop(0, nchunks_w, body, 0, unroll=False)

    plsc.subcore_barrier()

    @pl.when(s < NS - 1)
    def _():
        pltpu.sync_copy(acc.at[pl.ds(rbase, ROWS_A)],
                        out_hbm.at[c, pl.ds(rbase, ROWS_A)])

    @pl.when(s == NS - 1)
    def _():
        pltpu.sync_copy(acc.at[pl.ds((NS - 1) * ROWS_A, ROWS_LAST)],
                        out_hbm.at[c, pl.ds((NS - 1) * ROWS_A, ROWS_LAST)])


def _sc_segment_sum(idx3d, attr4):
    zeros = jnp.zeros((N, D_EDGE), dtype=jnp.float32)
    mesh = plsc.VectorSubcoreMesh(core_axis_name="c", subcore_axis_name="s")
    k = pl.kernel(
        _seg_sum_body,
        out_type=jax.ShapeDtypeStruct((NC, N, D_EDGE), jnp.float32),
        mesh=mesh,
        scratch_types=[
            pltpu.VMEM_SHARED((N, D_EDGE), jnp.float32),
            pltpu.VMEM((2, KSUB, SUB), jnp.int32),
            pltpu.VMEM((2, KSUB * 8, EB), jnp.float32),
            pltpu.VMEM((2, KSUB * 8, EB), jnp.float32),
            pltpu.VMEM((CHUNK, D_EDGE), jnp.float32),
            pltpu.SemaphoreType.DMA,
            pltpu.SemaphoreType.DMA,
            pltpu.SemaphoreType.DMA,
            pltpu.SemaphoreType.DMA,
        ],
        compiler_params=pltpu.CompilerParams(use_tc_tiling_on_sc=False,
                                             needs_layout_passes=False),
    )
    return k(zeros, idx3d, attr4)


BN = 2000


def _mlp_body(x_ref, p_ref, w1a_ref, w1b_ref, b1_ref, w2_ref, b2_ref, o_ref):
    agg = p_ref[0] + p_ref[1]
    h = jnp.dot(x_ref[...], w1a_ref[...], preferred_element_type=jnp.float32)
    h += jnp.dot(agg, w1b_ref[...], preferred_element_type=jnp.float32)
    h = jnp.maximum(h + b1_ref[...], 0.0)
    o = jnp.dot(h, w2_ref[...], preferred_element_type=jnp.float32)
    o_ref[...] = o + b2_ref[...]


def _tc_mlp(node_feats, partials, W1, b1, W2, b2):
    d_feat = node_feats.shape[1]
    w1a = W1[:, :d_feat].T
    w1b = W1[:, d_feat:].T
    w2 = W2.T
    b1r = b1.reshape(1, -1)
    b2r = b2.reshape(1, -1)
    grid = (N // BN,)
    return pl.pallas_call(
        _mlp_body,
        grid=grid,
        in_specs=[
            pl.BlockSpec((BN, d_feat), lambda i: (i, 0)),
            pl.BlockSpec((NC, BN, D_EDGE), lambda i: (0, i, 0)),
            pl.BlockSpec(w1a.shape, lambda i: (0, 0)),
            pl.BlockSpec(w1b.shape, lambda i: (0, 0)),
            pl.BlockSpec((1, b1r.shape[1]), lambda i: (0, 0)),
            pl.BlockSpec(w2.shape, lambda i: (0, 0)),
            pl.BlockSpec((1, b2r.shape[1]), lambda i: (0, 0)),
        ],
        out_specs=pl.BlockSpec((BN, W2.shape[0]), lambda i: (i, 0)),
        out_shape=jax.ShapeDtypeStruct((N, W2.shape[0]), jnp.float32),
    )(node_feats, partials, w1a, w1b, b1r, w2, b2r)


def kernel(node_feats, edge_index, edge_attr, W1, b1, W2, b2):
    idx3d = edge_index[0].reshape(NCHUNKS, KSUB, SUB)
    attr4 = edge_attr.T.reshape(NC, 8, NEB, EB).transpose(0, 2, 1, 3)
    attr3 = attr4.reshape(NC, NEB * 8, EB)
    partials = _sc_segment_sum(idx3d, attr3)
    return _tc_mlp(node_feats, partials, W1, b1, W2, b2)


# speedup vs baseline: 17.6152x; 1.1219x over previous
import functools

import jax
import jax.numpy as jnp
from jax import lax
from jax.experimental import pallas as pl
from jax.experimental.pallas import tpu as pltpu
from jax.experimental.pallas import tpu_sc as plsc

N = 100000
E = 3200000
D_EDGE = 16

NC = 2
NS = 16
NW = NC * NS

EB = 128
NEB = E // EB
SUB = 128
KSUB = 2
CHUNK = KSUB * EB
NCHUNKS = E // CHUNK
L = 16

ROWS_A = 6256
ROWS_LAST = N - (NS - 1) * ROWS_A

_BASE_CHUNKS = NCHUNKS // NW
_EXTRA_WORKERS = NCHUNKS % NW


def _seg_sum_body(zeros_hbm, idx_hbm, attr_hbm, out_hbm, acc, idxbuf, vlo, vhi,
                  rowbuf, sem_i, sem_lo, sem_hi, sem_s):
    c = lax.axis_index("c")
    s = lax.axis_index("s")
    wid = s * NC + c

    rbase = s * ROWS_A

    @pl.when(s < NS - 1)
    def _():
        pltpu.sync_copy(zeros_hbm.at[pl.ds(rbase, ROWS_A)],
                        acc.at[pl.ds(rbase, ROWS_A)])

    @pl.when(s == NS - 1)
    def _():
        pltpu.sync_copy(zeros_hbm.at[pl.ds((NS - 1) * ROWS_A, ROWS_LAST)],
                        acc.at[pl.ds((NS - 1) * ROWS_A, ROWS_LAST)])

    plsc.subcore_barrier()

    nchunks_w = _BASE_CHUNKS + jnp.where(wid < _EXTRA_WORKERS, 1, 0)
    iota16 = lax.iota(jnp.int32, L)

    def start_loads(j):
        chunk = wid + NW * j
        r0 = chunk * KSUB * 8
        b4 = lax.rem(j, 4)
        pltpu.async_copy(idx_hbm.at[chunk], idxbuf.at[lax.rem(j, 8)], sem_i)
        pltpu.async_copy(attr_hbm.at[0, pl.ds(r0, KSUB * 8)], vlo.at[b4],
                         sem_lo)
        pltpu.async_copy(attr_hbm.at[1, pl.ds(r0, KSUB * 8)], vhi.at[b4],
                         sem_hi)

    def wait_loads(j):
        chunk = wid + NW * j
        r0 = chunk * KSUB * 8
        b4 = lax.rem(j, 4)
        pltpu.make_async_copy(idx_hbm.at[chunk], idxbuf.at[lax.rem(j, 8)],
                              sem_i).wait()
        pltpu.make_async_copy(attr_hbm.at[0, pl.ds(r0, KSUB * 8)], vlo.at[b4],
                              sem_lo).wait()
        pltpu.make_async_copy(attr_hbm.at[1, pl.ds(r0, KSUB * 8)], vhi.at[b4],
                              sem_hi).wait()

    PF = 3
    for jj in range(PF):
        @pl.when(jj < nchunks_w)
        def _(jj=jj):
            start_loads(jj)

    def drain_scatters(jj):
        bb = lax.rem(jj, 2)
        for k in range(KSUB):
            pltpu.make_async_copy(
                rowbuf.at[bb, pl.ds(k * SUB, SUB)],
                acc.at[idxbuf.at[lax.rem(jj, 8), k]],
                sem_s).wait()

    def body(j, carry):
        b = lax.rem(j, 2)
        b4 = lax.rem(j, 4)
        bi = lax.rem(j, 8)
        wait_loads(j)

        @pl.when(j + PF < nchunks_w)
        def _():
            start_loads(j + PF)

        @pl.when(j >= 2)
        def _():
            drain_scatters(j - 2)

        for sb in range(KSUB):
            ebase = sb * EB
            for fi in range(8):
                fvec_lo = jnp.full((L,), fi, dtype=jnp.int32)
                fvec_hi = jnp.full((L,), fi + 8, dtype=jnp.int32)
                vs_lo = [vlo[b4, sb * 8 + fi, pl.ds(g * L, L)]
                         for g in range(EB // L)]
                vs_hi = [vhi[b4, sb * 8 + fi, pl.ds(g * L, L)]
                         for g in range(EB // L)]
                for g in range(EB // L):
                    evec = iota16 + (ebase + g * L)
                    plsc.store_scatter(rowbuf.at[b], [evec, fvec_lo], vs_lo[g])
                    plsc.store_scatter(rowbuf.at[b], [evec, fvec_hi], vs_hi[g])

        for k in range(KSUB):
            pltpu.async_copy(
                rowbuf.at[b, pl.ds(k * SUB, SUB)],
                acc.at[idxbuf.at[bi, k]],
                sem_s, add=True)
        return carry

    lax.fori_loop(0, nchunks_w, body, 0, unroll=False)
    drain_scatters(nchunks_w - 2)
    drain_scatters(nchunks_w - 1)

    plsc.subcore_barrier()

    @pl.when(s < NS - 1)
    def _():
        pltpu.sync_copy(acc.at[pl.ds(rbase, ROWS_A)],
                        out_hbm.at[c, pl.ds(rbase, ROWS_A)])

    @pl.when(s == NS - 1)
    def _():
        pltpu.sync_copy(acc.at[pl.ds((NS - 1) * ROWS_A, ROWS_LAST)],
                        out_hbm.at[c, pl.ds((NS - 1) * ROWS_A, ROWS_LAST)])


def _sc_segment_sum(idx3d, attr4):
    zeros = jnp.zeros((N, D_EDGE), dtype=jnp.float32)
    mesh = plsc.VectorSubcoreMesh(core_axis_name="c", subcore_axis_name="s")
    k = pl.kernel(
        _seg_sum_body,
        out_type=jax.ShapeDtypeStruct((NC, N, D_EDGE), jnp.float32),
        mesh=mesh,
        scratch_types=[
            pltpu.VMEM_SHARED((N, D_EDGE), jnp.float32),
            pltpu.VMEM((8, KSUB, SUB), jnp.int32),
            pltpu.VMEM((4, KSUB * 8, EB), jnp.float32),
            pltpu.VMEM((4, KSUB * 8, EB), jnp.float32),
            pltpu.VMEM((2, CHUNK, D_EDGE), jnp.float32),
            pltpu.SemaphoreType.DMA,
            pltpu.SemaphoreType.DMA,
            pltpu.SemaphoreType.DMA,
            pltpu.SemaphoreType.DMA,
        ],
        compiler_params=pltpu.CompilerParams(use_tc_tiling_on_sc=False,
                                             needs_layout_passes=False),
    )
    return k(zeros, idx3d, attr4)


BN = 2000


def _mlp_body(x_ref, p_ref, w1a_ref, w1b_ref, b1_ref, w2_ref, b2_ref, o_ref):
    agg = p_ref[0] + p_ref[1]
    h = jnp.dot(x_ref[...], w1a_ref[...], preferred_element_type=jnp.float32)
    h += jnp.dot(agg, w1b_ref[...], preferred_element_type=jnp.float32)
    h = jnp.maximum(h + b1_ref[...], 0.0)
    o = jnp.dot(h, w2_ref[...], preferred_element_type=jnp.float32)
    o_ref[...] = o + b2_ref[...]


def _tc_mlp(node_feats, partials, W1, b1, W2, b2):
    d_feat = node_feats.shape[1]
    w1a = W1[:, :d_feat].T
    w1b = W1[:, d_feat:].T
    w2 = W2.T
    b1r = b1.reshape(1, -1)
    b2r = b2.reshape(1, -1)
    grid = (N // BN,)
    return pl.pallas_call(
        _mlp_body,
        grid=grid,
        in_specs=[
            pl.BlockSpec((BN, d_feat), lambda i: (i, 0)),
            pl.BlockSpec((NC, BN, D_EDGE), lambda i: (0, i, 0)),
            pl.BlockSpec(w1a.shape, lambda i: (0, 0)),
            pl.BlockSpec(w1b.shape, lambda i: (0, 0)),
            pl.BlockSpec((1, b1r.shape[1]), lambda i: (0, 0)),
            pl.BlockSpec(w2.shape, lambda i: (0, 0)),
            pl.BlockSpec((1, b2r.shape[1]), lambda i: (0, 0)),
        ],
        out_specs=pl.BlockSpec((BN, W2.shape[0]), lambda i: (i, 0)),
        out_shape=jax.ShapeDtypeStruct((N, W2.shape[0]), jnp.float32),
    )(node_feats, partials, w1a, w1b, b1r, w2, b2r)


def kernel(node_feats, edge_index, edge_attr, W1, b1, W2, b2):
    idx3d = edge_index[0].reshape(NCHUNKS, KSUB, SUB)
    attr4 = edge_attr.T.reshape(NC, 8, NEB, EB).transpose(0, 2, 1, 3)
    attr3 = attr4.reshape(NC, NEB * 8, EB)
    partials = _sc_segment_sum(idx3d, attr3)
    return _tc_mlp(node_feats, partials, W1, b1, W2, b2)
